# baseline (speedup 1.0000x reference)
import jax
import jax.numpy as jnp
from jax import lax
from jax.experimental import pallas as pl
from jax.experimental.pallas import tpu as pltpu
from jax.experimental.pallas import tpu_sc as plsc

_B = 16384
_DIM = 64
_LANES = 16
_V = 1000000

_info = plsc.get_sparse_core_info()
_NC, _NS = _info.num_cores, _info.num_subcores
_NW = _NC * _NS
_BPW = _B // _NW
_CH = 128
_NR = _BPW // _CH
_TCB = 512


def _tc_body(a_ref, out_ref):
    ident = (lax.broadcasted_iota(jnp.int32, (_DIM, _DIM), 0)
             == lax.broadcasted_iota(jnp.int32, (_DIM, _DIM), 1)
             ).astype(jnp.float32)
    dn = (((0,), (0,)), ((), ()))
    y = lax.dot_general(a_ref[...], ident, dn,
                        precision=lax.Precision.HIGHEST,
                        preferred_element_type=jnp.float32)
    out_ref[:, 0:_DIM] = y
    out_ref[:, _DIM:2 * _DIM] = y


def _sc_body(target_hbm, context_hbm, table_hbm, out_hbm,
             idx_t, idx_c, u_bufs, v_bufs, out_v, sem):
    wid = lax.axis_index("s") * _NC + lax.axis_index("c")
    base = wid * _BPW

    for k in range(_NR):
        pltpu.sync_copy(target_hbm.at[pl.ds(base + k * _CH, _CH)],
                        idx_t.at[k])
        pltpu.sync_copy(context_hbm.at[pl.ds(base + k * _CH, _CH)],
                        idx_c.at[k])

    def fire(r):
        return (pltpu.async_copy(table_hbm.at[idx_t.at[r]],
                                 u_bufs.at[r % 2], sem),
                pltpu.async_copy(table_hbm.at[idx_c.at[r]],
                                 v_bufs.at[r % 2], sem))

    iota = lax.iota(jnp.int32, _LANES)
    inflight = fire(0)
    for r in range(_NR):
        for c in inflight:
            c.wait()
        if r + 1 < _NR:
            nxt = fire(r + 1)
        u_b, v_b = u_bufs.at[r % 2], v_bufs.at[r % 2]

        def group(g, _):
            rows = g * _LANES + iota
            acc = jnp.zeros((_LANES,), jnp.float32)
            for j in range(_DIM):
                col = jnp.full((_LANES,), j, jnp.int32)
                ug = plsc.load_gather(u_b, [rows, col])
                vg = plsc.load_gather(v_b, [rows, col])
                acc = acc + ug * vg
            out_v[pl.ds(r * _CH + g * _LANES, _LANES)] = acc
            return 0

        lax.fori_loop(0, _CH // _LANES, group, 0)
        if r + 1 < _NR:
            inflight = nxt

    pltpu.sync_copy(out_v, out_hbm.at[pl.ds(base, _BPW)])


@jax.jit
def kernel(target, context, embedding_weights):
    tt = embedding_weights.T
    n_blocks = (_V + _TCB - 1) // _TCB
    packed = pl.pallas_call(
        _tc_body,
        grid=(n_blocks,),
        in_specs=[pl.BlockSpec((_DIM, _TCB), lambda j: (0, j))],
        out_specs=pl.BlockSpec((_TCB, 2 * _DIM), lambda j: (j, 0)),
        out_shape=jax.ShapeDtypeStruct((_V, 2 * _DIM), jnp.float32),
    )(tt)

    mesh = plsc.VectorSubcoreMesh(core_axis_name="c", subcore_axis_name="s")
    run = pl.kernel(
        _sc_body,
        out_type=jax.ShapeDtypeStruct((_B,), jnp.float32),
        mesh=mesh,
        compiler_params=pltpu.CompilerParams(needs_layout_passes=False),
        scratch_types=[
            pltpu.VMEM((_NR, _CH), jnp.int32),
            pltpu.VMEM((_NR, _CH), jnp.int32),
            pltpu.VMEM((2, _CH, 2 * _DIM), jnp.float32),
            pltpu.VMEM((2, _CH, 2 * _DIM), jnp.float32),
            pltpu.VMEM((_BPW,), jnp.float32),
            pltpu.SemaphoreType.DMA,
        ],
    )
    return run(target.astype(jnp.int32), context.astype(jnp.int32), packed)
